# Initial kernel scaffold; baseline (speedup 1.0000x reference)
import jax
import jax.numpy as jnp
from jax.experimental import pallas as pl


def kernel(child_buffer, child_count, subs):
    raise NotImplementedError("write your pallas kernel here")


# speedup vs baseline: 11.2030x; 11.2030x over previous
import functools

import jax
import jax.numpy as jnp
from jax import lax
from jax.experimental import pallas as pl
from jax.experimental.pallas import tpu as pltpu
from jax.experimental.pallas import tpu_sc as plsc

L = 16


def _build_sc_call(B, N, MO, D):
    SLOTS = B * N
    ROWS = SLOTS * MO
    NW = 32
    SPW = SLOTS // NW
    CHUNK_SLOTS = 4
    CHUNK_ROWS = CHUNK_SLOTS * MO
    CHUNKS = SPW // CHUNK_SLOTS

    mesh = plsc.VectorSubcoreMesh(core_axis_name="c", subcore_axis_name="s")

    @functools.partial(
        pl.kernel,
        mesh=mesh,
        out_type=(
            jax.ShapeDtypeStruct((ROWS, D), jnp.float32),
            jax.ShapeDtypeStruct((SLOTS,), jnp.float32),
        ),
        scratch_types=[
            pltpu.VMEM((SPW,), jnp.float32),
            pltpu.VMEM((SPW,), jnp.int32),
            pltpu.VMEM((SPW,), jnp.float32),
            pltpu.VMEM((SPW,), jnp.int32),
            pltpu.VMEM((SPW,), jnp.int32),
            pltpu.VMEM((CHUNK_ROWS,), jnp.int32),
            pltpu.VMEM((CHUNK_ROWS, D), jnp.float32),
            pltpu.SemaphoreType.DMA,
        ],
    )
    def sc_fn(cb_hbm, cnt_hbm, subs_hbm, out_hbm, ncnt_hbm,
              cnt_v, subs_v, ncnt_v, cc_v, nv_v, idx_v, data_v, gsem):
        wid = lax.axis_index("s") * 2 + lax.axis_index("c")
        base_slot = wid * SPW

        pltpu.sync_copy(cnt_hbm.at[pl.ds(base_slot, SPW)], cnt_v)
        pltpu.sync_copy(subs_hbm.at[pl.ds(base_slot, SPW)], subs_v)

        lane = jnp.arange(L, dtype=jnp.int32)
        zrow = jnp.zeros((L,), jnp.float32)

        for g in range(SPW // L):
            cnt = cnt_v[pl.ds(g * L, L)]
            sb = subs_v[pl.ds(g * L, L)]
            rep = jnp.clip((sb + 2).astype(jnp.float32), 1.0, 3.0)
            nc = jnp.minimum(cnt * rep, float(MO))
            ncnt_v[pl.ds(g * L, L)] = nc
            fi = cnt.astype(jnp.int32)
            fr = cnt - fi.astype(jnp.float32)
            odd = lax.rem(fi, 2) == 1
            up = (fr > 0.5) | ((fr == 0.5) & odd)
            cc = jnp.maximum(fi + jnp.where(up, 1, 0), 1)
            cc_v[pl.ds(g * L, L)] = cc
            nci = nc.astype(jnp.int32)
            nv = nci + jnp.where(nci.astype(jnp.float32) < nc, 1, 0)
            nv_v[pl.ds(g * L, L)] = nv

        pltpu.sync_copy(ncnt_v, ncnt_hbm.at[pl.ds(base_slot, SPW)])

        def group_body(g, _):
            gs = base_slot + g * L
            cc_vec = cc_v[pl.ds(g * L, L)]
            nv_vec = nv_v[pl.ds(g * L, L)]
            for cq in range(L // CHUNK_SLOTS):
                for k in range(CHUNK_SLOTS):
                    j = cq * CHUNK_SLOTS + k
                    gbase = (gs + j) * MO
                    cc_b = jnp.broadcast_to(cc_vec[j], (L,))
                    for h in range(MO // L):
                        pos = lane + h * L
                        idx_v[pl.ds(k * MO + h * L, L)] = (
                            gbase + lax.rem(pos, cc_b))
                pltpu.async_copy(cb_hbm.at[idx_v], data_v, gsem).wait()
                for k in range(CHUNK_SLOTS):
                    j = cq * CHUNK_SLOTS + k
                    nv_s = nv_vec[j]

                    def zbody(p, _, _k=k):
                        for jj in range(D // L):
                            data_v[_k * MO + p, pl.ds(jj * L, L)] = zrow
                        return 0

                    lax.fori_loop(nv_s, MO, zbody, 0)
                pltpu.sync_copy(
                    data_v,
                    out_hbm.at[pl.ds((gs + cq * CHUNK_SLOTS) * MO,
                                     CHUNK_ROWS)])
            return 0

        lax.fori_loop(0, SPW // L, group_body, 0)

    return sc_fn


def kernel(child_buffer, child_count, subs):
    b, n, mo, d = child_buffer.shape
    fn = _build_sc_call(b, n, mo, d)
    out, ncnt = fn(
        child_buffer.reshape(b * n * mo, d),
        child_count.reshape(b * n),
        subs.reshape(b * n),
    )
    return out.reshape(b, n, mo, d), ncnt.reshape(b, n)


# speedup vs baseline: 11.8708x; 1.0596x over previous
import functools

import jax
import jax.numpy as jnp
from jax import lax
from jax.experimental import pallas as pl
from jax.experimental.pallas import tpu as pltpu
from jax.experimental.pallas import tpu_sc as plsc

L = 16


def _build_sc_call(B, N, MO, D):
    SLOTS = B * N
    ROWS = SLOTS * MO
    NW = 32
    SPW = SLOTS // NW
    CHUNK_SLOTS = 4
    CHUNK_ROWS = CHUNK_SLOTS * MO
    GROUPS = SPW // L
    CPG = L // CHUNK_SLOTS

    mesh = plsc.VectorSubcoreMesh(core_axis_name="c", subcore_axis_name="s")

    @functools.partial(
        pl.kernel,
        mesh=mesh,
        out_type=(
            jax.ShapeDtypeStruct((ROWS, D), jnp.float32),
            jax.ShapeDtypeStruct((SLOTS,), jnp.float32),
        ),
        scratch_types=[
            pltpu.VMEM((SPW,), jnp.float32),
            pltpu.VMEM((SPW,), jnp.int32),
            pltpu.VMEM((SPW,), jnp.float32),
            pltpu.VMEM((SPW,), jnp.int32),
            pltpu.VMEM((SPW * MO,), jnp.int32),
            pltpu.VMEM((CHUNK_ROWS, D), jnp.float32),
            pltpu.VMEM((CHUNK_ROWS, D), jnp.float32),
            pltpu.SemaphoreType.DMA,
            pltpu.SemaphoreType.DMA,
            pltpu.SemaphoreType.DMA,
            pltpu.SemaphoreType.DMA,
        ],
    )
    def sc_fn(cb_hbm, cnt_hbm, subs_hbm, out_hbm, ncnt_hbm,
              cnt_v, subs_v, ncnt_v, nv_v, idx_v, buf0, buf1,
              gs0, gs1, ws0, ws1):
        wid = lax.axis_index("s") * 2 + lax.axis_index("c")
        base_slot = wid * SPW

        pltpu.sync_copy(cnt_hbm.at[pl.ds(base_slot, SPW)], cnt_v)
        pltpu.sync_copy(subs_hbm.at[pl.ds(base_slot, SPW)], subs_v)

        lane = jnp.arange(L, dtype=jnp.int32)
        zrow = jnp.zeros((L,), jnp.float32)
        bufs = (buf0, buf1)
        gsems = (gs0, gs1)
        wsems = (ws0, ws1)

        def meta_body(g, _):
            cnt = cnt_v[pl.ds(g * L, L)]
            sb = subs_v[pl.ds(g * L, L)]
            rep = jnp.clip((sb + 2).astype(jnp.float32), 1.0, 3.0)
            nc = jnp.minimum(cnt * rep, float(MO))
            ncnt_v[pl.ds(g * L, L)] = nc
            fi = cnt.astype(jnp.int32)
            fr = cnt - fi.astype(jnp.float32)
            odd = lax.rem(fi, 2) == 1
            up = (fr > 0.5) | ((fr == 0.5) & odd)
            cc = jnp.maximum(fi + jnp.where(up, 1, 0), 1)
            nci = nc.astype(jnp.int32)
            nv = nci + jnp.where(nci.astype(jnp.float32) < nc, 1, 0)
            nv_v[pl.ds(g * L, L)] = nv
            for j in range(L):
                cc_b = jnp.broadcast_to(cc[j], (L,))
                gbase = (base_slot + g * L + j) * MO
                for h in range(MO // L):
                    idx_v[pl.ds((g * L + j) * MO + h * L, L)] = (
                        gbase + lax.rem(lane + h * L, cc_b))
            return 0

        lax.fori_loop(0, GROUPS, meta_body, 0)
        pltpu.sync_copy(ncnt_v, ncnt_hbm.at[pl.ds(base_slot, SPW)])

        def gather_start(c, par):
            pltpu.async_copy(
                cb_hbm.at[idx_v.at[pl.ds(c * CHUNK_ROWS, CHUNK_ROWS)]],
                bufs[par], gsems[par])

        def gather_wait(par):
            pltpu.make_async_copy(
                cb_hbm.at[pl.ds(0, CHUNK_ROWS)], bufs[par],
                gsems[par]).wait()

        def write_start(c, par):
            pltpu.async_copy(
                bufs[par],
                out_hbm.at[pl.ds((base_slot + c * CHUNK_SLOTS) * MO,
                                 CHUNK_ROWS)],
                wsems[par])

        def write_wait(par):
            pltpu.make_async_copy(
                bufs[par], out_hbm.at[pl.ds(0, CHUNK_ROWS)],
                wsems[par]).wait()

        gather_start(0, 0)

        def group_body(g, _):
            nv_vec = nv_v[pl.ds(g * L, L)]
            for cq in range(CPG):
                c = g * CPG + cq
                par = cq % 2
                gather_wait(par)
                if cq == 0:
                    @pl.when(g >= 1)
                    def _():
                        write_wait(par ^ 1)
                else:
                    write_wait(par ^ 1)
                if cq == CPG - 1:
                    @pl.when(g < GROUPS - 1)
                    def _():
                        gather_start(c + 1, par ^ 1)
                else:
                    gather_start(c + 1, par ^ 1)
                for k in range(CHUNK_SLOTS):
                    nv_s = nv_vec[cq * CHUNK_SLOTS + k]

                    def zbody(p, _, _k=k, _par=par):
                        for jj in range(D // L):
                            bufs[_par][_k * MO + p, pl.ds(jj * L, L)] = zrow
                        return 0

                    lax.fori_loop(nv_s, MO, zbody, 0)
                write_start(c, par)
            return 0

        lax.fori_loop(0, GROUPS, group_body, 0)
        write_wait(1)

    return sc_fn


def kernel(child_buffer, child_count, subs):
    b, n, mo, d = child_buffer.shape
    fn = _build_sc_call(b, n, mo, d)
    out, ncnt = fn(
        child_buffer.reshape(b * n * mo, d),
        child_count.reshape(b * n),
        subs.reshape(b * n),
    )
    return out.reshape(b, n, mo, d), ncnt.reshape(b, n)


# speedup vs baseline: 14.4337x; 1.2159x over previous
import functools

import jax
import jax.numpy as jnp
from jax import lax
from jax.experimental import pallas as pl
from jax.experimental.pallas import tpu as pltpu
from jax.experimental.pallas import tpu_sc as plsc

L = 16


def _build_sc_call(B, N, MO, D):
    SLOTS = B * N
    ROWS = SLOTS * MO
    NW = 32
    SPW = SLOTS // NW
    CHUNK_SLOTS = 4
    CHUNK_ROWS = CHUNK_SLOTS * MO
    GROUPS = SPW // L
    CPG = L // CHUNK_SLOTS
    RQ = MO // 8

    mesh = plsc.VectorSubcoreMesh(core_axis_name="c", subcore_axis_name="s")

    @functools.partial(
        pl.kernel,
        mesh=mesh,
        out_type=(
            jax.ShapeDtypeStruct((ROWS, D), jnp.float32),
            jax.ShapeDtypeStruct((SLOTS,), jnp.float32),
        ),
        scratch_types=[
            pltpu.VMEM((SPW,), jnp.float32),
            pltpu.VMEM((SPW,), jnp.int32),
            pltpu.VMEM((SPW,), jnp.float32),
            pltpu.VMEM((SPW,), jnp.int32),
            pltpu.VMEM((SPW,), jnp.int32),
            pltpu.VMEM((CHUNK_ROWS, D), jnp.float32),
            pltpu.VMEM((CHUNK_ROWS, D), jnp.float32),
            pltpu.SemaphoreType.DMA,
            pltpu.SemaphoreType.DMA,
            pltpu.SemaphoreType.DMA,
            pltpu.SemaphoreType.DMA,
        ],
    )
    def sc_fn(cb_hbm, cnt_hbm, subs_hbm, out_hbm, ncnt_hbm,
              cnt_v, subs_v, ncnt_v, cc_v, nv_v, buf0, buf1,
              rs0, rs1, ws0, ws1):
        wid = lax.axis_index("s") * 2 + lax.axis_index("c")
        base_slot = wid * SPW

        pltpu.sync_copy(cnt_hbm.at[pl.ds(base_slot, SPW)], cnt_v)
        pltpu.sync_copy(subs_hbm.at[pl.ds(base_slot, SPW)], subs_v)

        zrow = jnp.zeros((L,), jnp.float32)
        bufs = (buf0, buf1)
        rsems = (rs0, rs1)
        wsems = (ws0, ws1)

        def meta_body(g, _):
            cnt = cnt_v[pl.ds(g * L, L)]
            sb = subs_v[pl.ds(g * L, L)]
            rep = jnp.clip((sb + 2).astype(jnp.float32), 1.0, 3.0)
            nc = jnp.minimum(cnt * rep, float(MO))
            ncnt_v[pl.ds(g * L, L)] = nc
            fi = cnt.astype(jnp.int32)
            fr = cnt - fi.astype(jnp.float32)
            odd = lax.rem(fi, 2) == 1
            up = (fr > 0.5) | ((fr == 0.5) & odd)
            cc = jnp.maximum(fi + jnp.where(up, 1, 0), 1)
            cc_v[pl.ds(g * L, L)] = cc
            nci = nc.astype(jnp.int32)
            nv = nci + jnp.where(nci.astype(jnp.float32) < nc, 1, 0)
            nv_v[pl.ds(g * L, L)] = nv
            return 0

        lax.fori_loop(0, GROUPS, meta_body, 0)
        pltpu.sync_copy(ncnt_v, ncnt_hbm.at[pl.ds(base_slot, SPW)])

        def reads_start(g, cq, par, cc_vec):
            for k in range(CHUNK_SLOTS):
                cc_s = cc_vec[cq * CHUNK_SLOTS + k]
                srow = (base_slot + g * L + cq * CHUNK_SLOTS + k) * MO
                for q in range(RQ):
                    @pl.when(8 * q < cc_s)
                    def _():
                        pltpu.async_copy(
                            cb_hbm.at[pl.ds(srow + 8 * q, 8)],
                            bufs[par].at[pl.ds(k * MO + 8 * q, 8)],
                            rsems[par])

        def reads_wait(cq, par, cc_vec):
            for k in range(CHUNK_SLOTS):
                cc_s = cc_vec[cq * CHUNK_SLOTS + k]
                for q in range(RQ):
                    @pl.when(8 * q < cc_s)
                    def _():
                        pltpu.make_async_copy(
                            cb_hbm.at[pl.ds(0, 8)],
                            bufs[par].at[pl.ds(k * MO + 8 * q, 8)],
                            rsems[par]).wait()

        def write_start(g, cq, par):
            pltpu.async_copy(
                bufs[par],
                out_hbm.at[pl.ds((base_slot + g * L + cq * CHUNK_SLOTS) * MO,
                                 CHUNK_ROWS)],
                wsems[par])

        def write_wait(par):
            pltpu.make_async_copy(
                bufs[par], out_hbm.at[pl.ds(0, CHUNK_ROWS)],
                wsems[par]).wait()

        cc_vec0 = cc_v[pl.ds(0, L)]
        reads_start(jnp.int32(0), 0, 0, cc_vec0)

        def group_body(g, _):
            cc_vec = cc_v[pl.ds(g * L, L)]
            nv_vec = nv_v[pl.ds(g * L, L)]
            cc_vec_n = cc_v[pl.ds(jnp.minimum(g + 1, GROUPS - 1) * L, L)]
            for cq in range(CPG):
                c = g * CPG + cq
                par = cq % 2
                reads_wait(cq, par, cc_vec)
                if cq == 0:
                    @pl.when(g >= 1)
                    def _():
                        write_wait(par ^ 1)
                else:
                    write_wait(par ^ 1)
                if cq == CPG - 1:
                    @pl.when(g < GROUPS - 1)
                    def _():
                        reads_start(g + 1, 0, par ^ 1, cc_vec_n)
                else:
                    reads_start(g, cq + 1, par ^ 1, cc_vec)
                for k in range(CHUNK_SLOTS):
                    cc_s = cc_vec[cq * CHUNK_SLOTS + k]
                    nv_s = nv_vec[cq * CHUNK_SLOTS + k]

                    def rbody(p, _, _k=k, _par=par, _cc=cc_s):
                        for jj in range(D // L):
                            bufs[_par][_k * MO + p, pl.ds(jj * L, L)] = (
                                bufs[_par][_k * MO + p - _cc,
                                           pl.ds(jj * L, L)])
                        return 0

                    lax.fori_loop(cc_s, nv_s, rbody, 0)

                    def zbody(p, _, _k=k, _par=par):
                        for jj in range(D // L):
                            bufs[_par][_k * MO + p, pl.ds(jj * L, L)] = zrow
                        return 0

                    lax.fori_loop(nv_s, MO, zbody, 0)
                write_start(g, cq, par)
            return 0

        lax.fori_loop(0, GROUPS, group_body, 0)
        write_wait(1)

    return sc_fn


def kernel(child_buffer, child_count, subs):
    b, n, mo, d = child_buffer.shape
    fn = _build_sc_call(b, n, mo, d)
    out, ncnt = fn(
        child_buffer.reshape(b * n * mo, d),
        child_count.reshape(b * n),
        subs.reshape(b * n),
    )
    return out.reshape(b, n, mo, d), ncnt.reshape(b, n)


# speedup vs baseline: 16.3253x; 1.1311x over previous
import functools

import jax
import jax.numpy as jnp
from jax import lax
from jax.experimental import pallas as pl
from jax.experimental.pallas import tpu as pltpu
from jax.experimental.pallas import tpu_sc as plsc

L = 16


def _build_sc_call(B, N, MO, D):
    SLOTS = B * N
    ROWS = SLOTS * MO
    NW = 32
    SPW = SLOTS // NW
    CHUNK_SLOTS = 2
    CHUNK_ROWS = CHUNK_SLOTS * MO
    GROUPS = SPW // L
    CPG = L // CHUNK_SLOTS
    NBUF = 4
    RQ = MO // 8

    mesh = plsc.VectorSubcoreMesh(core_axis_name="c", subcore_axis_name="s")

    @functools.partial(
        pl.kernel,
        mesh=mesh,
        out_type=(
            jax.ShapeDtypeStruct((ROWS, D), jnp.float32),
            jax.ShapeDtypeStruct((SLOTS,), jnp.float32),
        ),
        scratch_types=(
            [
                pltpu.VMEM((SPW,), jnp.float32),
                pltpu.VMEM((SPW,), jnp.int32),
                pltpu.VMEM((SPW,), jnp.float32),
                pltpu.VMEM((SPW,), jnp.int32),
                pltpu.VMEM((SPW,), jnp.int32),
            ]
            + [pltpu.VMEM((CHUNK_ROWS, D), jnp.float32)] * NBUF
            + [pltpu.SemaphoreType.DMA] * (2 * NBUF)
        ),
    )
    def sc_fn(cb_hbm, cnt_hbm, subs_hbm, out_hbm, ncnt_hbm,
              cnt_v, subs_v, ncnt_v, cc_v, nv_v,
              b0, b1, b2, b3, r0, r1, r2, r3, w0, w1, w2, w3):
        wid = lax.axis_index("s") * 2 + lax.axis_index("c")
        base_slot = wid * SPW

        pltpu.sync_copy(cnt_hbm.at[pl.ds(base_slot, SPW)], cnt_v)
        pltpu.sync_copy(subs_hbm.at[pl.ds(base_slot, SPW)], subs_v)

        zrow = jnp.zeros((L,), jnp.float32)
        bufs = (b0, b1, b2, b3)
        rsems = (r0, r1, r2, r3)
        wsems = (w0, w1, w2, w3)

        def meta_body(g, _):
            cnt = cnt_v[pl.ds(g * L, L)]
            sb = subs_v[pl.ds(g * L, L)]
            rep = jnp.clip((sb + 2).astype(jnp.float32), 1.0, 3.0)
            nc = jnp.minimum(cnt * rep, float(MO))
            ncnt_v[pl.ds(g * L, L)] = nc
            fi = cnt.astype(jnp.int32)
            fr = cnt - fi.astype(jnp.float32)
            odd = lax.rem(fi, 2) == 1
            up = (fr > 0.5) | ((fr == 0.5) & odd)
            cc = jnp.maximum(fi + jnp.where(up, 1, 0), 1)
            cc_v[pl.ds(g * L, L)] = cc
            nci = nc.astype(jnp.int32)
            nv = nci + jnp.where(nci.astype(jnp.float32) < nc, 1, 0)
            nv_v[pl.ds(g * L, L)] = nv
            return 0

        lax.fori_loop(0, GROUPS, meta_body, 0)
        pltpu.sync_copy(ncnt_v, ncnt_hbm.at[pl.ds(base_slot, SPW)])

        def reads_start(g, cq, par, cc_vec):
            for k in range(CHUNK_SLOTS):
                cc_s = cc_vec[cq * CHUNK_SLOTS + k]
                srow = (base_slot + g * L + cq * CHUNK_SLOTS + k) * MO
                for q in range(RQ):
                    @pl.when(8 * q < cc_s)
                    def _():
                        pltpu.async_copy(
                            cb_hbm.at[pl.ds(srow + 8 * q, 8)],
                            bufs[par].at[pl.ds(k * MO + 8 * q, 8)],
                            rsems[par])

        def reads_wait(cq, par, cc_vec):
            for k in range(CHUNK_SLOTS):
                cc_s = cc_vec[cq * CHUNK_SLOTS + k]
                for q in range(RQ):
                    @pl.when(8 * q < cc_s)
                    def _():
                        pltpu.make_async_copy(
                            cb_hbm.at[pl.ds(0, 8)],
                            bufs[par].at[pl.ds(k * MO + 8 * q, 8)],
                            rsems[par]).wait()

        def write_start(g, cq, par):
            pltpu.async_copy(
                bufs[par],
                out_hbm.at[pl.ds((base_slot + g * L + cq * CHUNK_SLOTS) * MO,
                                 CHUNK_ROWS)],
                wsems[par])

        def write_wait(par):
            pltpu.make_async_copy(
                bufs[par], out_hbm.at[pl.ds(0, CHUNK_ROWS)],
                wsems[par]).wait()

        cc_vec0 = cc_v[pl.ds(0, L)]
        reads_start(jnp.int32(0), 0, 0, cc_vec0)

        def group_body(g, _):
            cc_vec = cc_v[pl.ds(g * L, L)]
            nv_vec = nv_v[pl.ds(g * L, L)]
            cc_vec_n = cc_v[pl.ds(jnp.minimum(g + 1, GROUPS - 1) * L, L)]
            for cq in range(CPG):
                par = cq % NBUF
                par1 = (cq + 1) % NBUF
                reads_wait(cq, par, cc_vec)
                if cq < NBUF - 1:
                    @pl.when(g >= 1)
                    def _():
                        write_wait(par1)
                else:
                    write_wait(par1)
                if cq == CPG - 1:
                    @pl.when(g < GROUPS - 1)
                    def _():
                        reads_start(g + 1, 0, par1, cc_vec_n)
                else:
                    reads_start(g, cq + 1, par1, cc_vec)
                for k in range(CHUNK_SLOTS):
                    cc_s = cc_vec[cq * CHUNK_SLOTS + k]
                    nv_s = nv_vec[cq * CHUNK_SLOTS + k]

                    def rbody(p, _, _k=k, _par=par, _cc=cc_s):
                        for jj in range(D // L):
                            bufs[_par][_k * MO + p, pl.ds(jj * L, L)] = (
                                bufs[_par][_k * MO + p - _cc,
                                           pl.ds(jj * L, L)])
                        return 0

                    lax.fori_loop(cc_s, nv_s, rbody, 0)

                    def zbody(p, _, _k=k, _par=par):
                        for jj in range(D // L):
                            bufs[_par][_k * MO + p, pl.ds(jj * L, L)] = zrow
                        return 0

                    lax.fori_loop(nv_s, MO, zbody, 0)
                write_start(g, cq, par)
            return 0

        lax.fori_loop(0, GROUPS, group_body, 0)
        write_wait((CPG * GROUPS - 3) % NBUF)
        write_wait((CPG * GROUPS - 2) % NBUF)
        write_wait((CPG * GROUPS - 1) % NBUF)

    return sc_fn


def kernel(child_buffer, child_count, subs):
    b, n, mo, d = child_buffer.shape
    fn = _build_sc_call(b, n, mo, d)
    out, ncnt = fn(
        child_buffer.reshape(b * n * mo, d),
        child_count.reshape(b * n),
        subs.reshape(b * n),
    )
    return out.reshape(b, n, mo, d), ncnt.reshape(b, n)
